# baseline (speedup 1.0000x reference)
import jax
import jax.numpy as jnp
from jax.experimental import pallas as pl
from jax.experimental.pallas import tpu as pltpu


def _prep_body(x_ref, w1_ref, u_ref, s_ref, sm_ref, sq_ref):
    x = x_ref[...]
    u = jnp.dot(x, w1_ref[...], preferred_element_type=jnp.float32)
    u_ref[...] = u.astype(jnp.bfloat16)
    rn = jnp.sqrt(jnp.sum(x * x, axis=1, keepdims=True))
    s_ref[...] = jnp.sum(x / (rn + 1e-8), axis=0)[None, None, :]
    sm_ref[...] = jnp.sum(x, axis=0)[None, None, :]
    sq_ref[...] = jnp.sum(x * x, axis=0)[None, None, :]


def _layer1_body(adj_ref, u_ref, b1_ref, w2_ref, v_ref, s_ref, sm_ref, sq_ref):
    a = adj_ref[...].astype(jnp.bfloat16)
    h1 = jnp.dot(a, u_ref[...], preferred_element_type=jnp.float32)
    h1 = h1 + b1_ref[...]
    rn = jnp.sqrt(jnp.sum(h1 * h1, axis=1, keepdims=True))
    s_ref[...] = jnp.sum(h1 / (rn + 1e-8), axis=0)[None, None, :]
    sm_ref[...] = jnp.sum(h1, axis=0)[None, None, :]
    sq_ref[...] = jnp.sum(h1 * h1, axis=0)[None, None, :]
    h = jnp.maximum(h1, 0.0).astype(jnp.bfloat16)
    v_ref[...] = jnp.dot(h, w2_ref[...], preferred_element_type=jnp.float32
                         ).astype(jnp.bfloat16)


def _layer2_body(adj_ref, v_ref, b2_ref, out_ref, s_ref, sm_ref, sq_ref):
    a = adj_ref[...].astype(jnp.bfloat16)
    o = jnp.dot(a, v_ref[...], preferred_element_type=jnp.float32)
    o = o + b2_ref[...]
    out_ref[...] = o
    rn = jnp.sqrt(jnp.sum(o * o, axis=1, keepdims=True))
    s_ref[...] = jnp.sum(o / (rn + 1e-8), axis=0)[None, None, :]
    sm_ref[...] = jnp.sum(o, axis=0)[None, None, :]
    sq_ref[...] = jnp.sum(o * o, axis=0)[None, None, :]


def _sim_from_s(s, n):
    total = jnp.dot(s, s)
    return (total - n) / (n * (n - 1))


def _var_from_partials(sm, sq, count):
    tot = jnp.sum(sm)
    totsq = jnp.sum(sq)
    mean = tot / count
    return totsq / count - mean * mean


def kernel(x, adj, W1, b1, W2, b2):
    n, nfeat = x.shape
    nhid = W1.shape[1]
    nclass = W2.shape[1]
    fdt = jnp.float32

    tm1 = 1000
    g1 = n // tm1
    u, s1p, sm1p, sq1p = pl.pallas_call(
        _prep_body,
        grid=(g1,),
        in_specs=[
            pl.BlockSpec((tm1, nfeat), lambda i: (i, 0)),
            pl.BlockSpec((nfeat, nhid), lambda i: (0, 0)),
        ],
        out_specs=[
            pl.BlockSpec((tm1, nhid), lambda i: (i, 0)),
            pl.BlockSpec((1, 1, nfeat), lambda i: (i, 0, 0)),
            pl.BlockSpec((1, 1, nfeat), lambda i: (i, 0, 0)),
            pl.BlockSpec((1, 1, nfeat), lambda i: (i, 0, 0)),
        ],
        out_shape=[
            jax.ShapeDtypeStruct((n, nhid), jnp.bfloat16),
            jax.ShapeDtypeStruct((g1, 1, nfeat), fdt),
            jax.ShapeDtypeStruct((g1, 1, nfeat), fdt),
            jax.ShapeDtypeStruct((g1, 1, nfeat), fdt),
        ],
        compiler_params=pltpu.CompilerParams(
            dimension_semantics=("parallel",)),
    )(x, W1)

    tm = 200
    g = n // tm
    b1r = b1.reshape(1, nhid)
    w2b = W2.astype(jnp.bfloat16)
    v, s2p, sm2p, sq2p = pl.pallas_call(
        _layer1_body,
        grid=(g,),
        in_specs=[
            pl.BlockSpec((tm, n), lambda i: (i, 0)),
            pl.BlockSpec((n, nhid), lambda i: (0, 0)),
            pl.BlockSpec((1, nhid), lambda i: (0, 0)),
            pl.BlockSpec((nhid, nclass), lambda i: (0, 0)),
        ],
        out_specs=[
            pl.BlockSpec((tm, nclass), lambda i: (i, 0)),
            pl.BlockSpec((1, 1, nhid), lambda i: (i, 0, 0)),
            pl.BlockSpec((1, 1, nhid), lambda i: (i, 0, 0)),
            pl.BlockSpec((1, 1, nhid), lambda i: (i, 0, 0)),
        ],
        out_shape=[
            jax.ShapeDtypeStruct((n, nclass), jnp.bfloat16),
            jax.ShapeDtypeStruct((g, 1, nhid), fdt),
            jax.ShapeDtypeStruct((g, 1, nhid), fdt),
            jax.ShapeDtypeStruct((g, 1, nhid), fdt),
        ],
        compiler_params=pltpu.CompilerParams(
            dimension_semantics=("parallel",)),
    )(adj, u, b1r, w2b)

    b2r = b2.reshape(1, nclass)
    out, s4p, sm4p, sq4p = pl.pallas_call(
        _layer2_body,
        grid=(g,),
        in_specs=[
            pl.BlockSpec((tm, n), lambda i: (i, 0)),
            pl.BlockSpec((n, nclass), lambda i: (0, 0)),
            pl.BlockSpec((1, nclass), lambda i: (0, 0)),
        ],
        out_specs=[
            pl.BlockSpec((tm, nclass), lambda i: (i, 0)),
            pl.BlockSpec((1, 1, nclass), lambda i: (i, 0, 0)),
            pl.BlockSpec((1, 1, nclass), lambda i: (i, 0, 0)),
            pl.BlockSpec((1, 1, nclass), lambda i: (i, 0, 0)),
        ],
        out_shape=[
            jax.ShapeDtypeStruct((n, nclass), fdt),
            jax.ShapeDtypeStruct((g, 1, nclass), fdt),
            jax.ShapeDtypeStruct((g, 1, nclass), fdt),
            jax.ShapeDtypeStruct((g, 1, nclass), fdt),
        ],
        compiler_params=pltpu.CompilerParams(
            dimension_semantics=("parallel",)),
    )(adj, v, b2r)

    nf = jnp.float32(n)
    sim1 = _sim_from_s(jnp.sum(s1p, axis=(0, 1)), nf)
    var1 = _var_from_partials(sm1p, sq1p, nf * nfeat)
    sim2 = _sim_from_s(jnp.sum(s2p, axis=(0, 1)), nf)
    var2 = _var_from_partials(sm2p, sq2p, nf * nhid)
    sim4 = _sim_from_s(jnp.sum(s4p, axis=(0, 1)), nf)
    var4 = _var_from_partials(sm4p, sq4p, nf * nclass)
    z = jnp.float32(0.0)
    metrics = jnp.stack([sim1, z, var1, sim2, z, var2, sim2, z, var2,
                         sim4, z, var4])
    return (out, metrics)


# speedup vs baseline: 1.0797x; 1.0797x over previous
import jax
import jax.numpy as jnp
from jax import lax
from jax.experimental import pallas as pl
from jax.experimental.pallas import tpu as pltpu


def _colstats(m):
    rn = jnp.sqrt(jnp.sum(m * m, axis=1, keepdims=True))
    s = jnp.sum(m / (rn + 1e-8), axis=0, keepdims=True)
    sm = jnp.sum(m, axis=0, keepdims=True)
    sq = jnp.sum(m * m, axis=0, keepdims=True)
    return jnp.concatenate([s, sm, sq], axis=0)


def _sim_var(stats, n, d):
    s = stats[0, :]
    sim = (jnp.sum(s * s) - n) / (n * (n - 1.0))
    cnt = n * d
    mean = jnp.sum(stats[1, :]) / cnt
    var = jnp.sum(stats[2, :]) / cnt - mean * mean
    return sim, var


def _layer1_body(adj_ref, x_ref, w1_ref, b1_ref, w2_ref,
                 v_ref, xst_ref, hst_ref, u_ref):
    i = pl.program_id(0)

    @pl.when(i == 0)
    def _prep():
        x = x_ref[...]
        u = jnp.dot(x, w1_ref[...], preferred_element_type=jnp.float32)
        u_ref[...] = u.astype(jnp.bfloat16)
        xst_ref[0] = _colstats(x)

    a = adj_ref[...].astype(jnp.bfloat16)
    h1 = jnp.dot(a, u_ref[...], preferred_element_type=jnp.float32)
    h1 = h1 + b1_ref[...]
    st = _colstats(h1)

    @pl.when(i == 0)
    def _init():
        hst_ref[0] = st

    @pl.when(i > 0)
    def _acc():
        hst_ref[0] += st

    h = jnp.maximum(h1, 0.0).astype(jnp.bfloat16)
    v_ref[...] = jnp.dot(h, w2_ref[...], preferred_element_type=jnp.float32
                         ).astype(jnp.bfloat16)


def _layer2_body(adj_ref, v_ref, b2_ref, xst_ref, hst_ref,
                 out_ref, m_ref, acc_ref):
    i = pl.program_id(0)
    ng = pl.num_programs(0)
    a = adj_ref[...].astype(jnp.bfloat16)
    o = jnp.dot(a, v_ref[...], preferred_element_type=jnp.float32)
    o = o + b2_ref[...]
    out_ref[...] = o
    st = _colstats(o)

    @pl.when(i == 0)
    def _init():
        acc_ref[...] = st

    @pl.when(i > 0)
    def _acc():
        acc_ref[...] += st

    @pl.when(i == ng - 1)
    def _finalize():
        n = jnp.float32(out_ref.shape[0] * ng)
        sim1, var1 = _sim_var(xst_ref[0], n, jnp.float32(xst_ref.shape[2]))
        sim2, var2 = _sim_var(hst_ref[0], n, jnp.float32(hst_ref.shape[2]))
        sim4, var4 = _sim_var(acc_ref[...], n, jnp.float32(acc_ref.shape[1]))
        lane = lax.broadcasted_iota(jnp.int32, (1, 16), 1)
        mv = jnp.zeros((1, 16), jnp.float32)
        for k, val in ((0, sim1), (2, var1), (3, sim2), (5, var2),
                       (6, sim2), (8, var2), (9, sim4), (11, var4)):
            mv = jnp.where(lane == k, val, mv)
        m_ref[...] = mv


def kernel(x, adj, W1, b1, W2, b2):
    n, nfeat = x.shape
    nhid = W1.shape[1]
    nclass = W2.shape[1]
    fdt = jnp.float32
    tm = 200
    g = n // tm

    v, xst, hst = pl.pallas_call(
        _layer1_body,
        grid=(g,),
        in_specs=[
            pl.BlockSpec((tm, n), lambda i: (i, 0)),
            pl.BlockSpec((n, nfeat), lambda i: (0, 0)),
            pl.BlockSpec((nfeat, nhid), lambda i: (0, 0)),
            pl.BlockSpec((1, nhid), lambda i: (0, 0)),
            pl.BlockSpec((nhid, nclass), lambda i: (0, 0)),
        ],
        out_specs=[
            pl.BlockSpec((tm, nclass), lambda i: (i, 0)),
            pl.BlockSpec((1, 3, nfeat), lambda i: (0, 0, 0)),
            pl.BlockSpec((1, 3, nhid), lambda i: (0, 0, 0)),
        ],
        out_shape=[
            jax.ShapeDtypeStruct((n, nclass), jnp.bfloat16),
            jax.ShapeDtypeStruct((1, 3, nfeat), fdt),
            jax.ShapeDtypeStruct((1, 3, nhid), fdt),
        ],
        scratch_shapes=[pltpu.VMEM((n, nhid), jnp.bfloat16)],
        compiler_params=pltpu.CompilerParams(
            dimension_semantics=("arbitrary",)),
    )(adj, x, W1, b1.reshape(1, nhid), W2.astype(jnp.bfloat16))

    out, mv = pl.pallas_call(
        _layer2_body,
        grid=(g,),
        in_specs=[
            pl.BlockSpec((tm, n), lambda i: (i, 0)),
            pl.BlockSpec((n, nclass), lambda i: (0, 0)),
            pl.BlockSpec((1, nclass), lambda i: (0, 0)),
            pl.BlockSpec((1, 3, nfeat), lambda i: (0, 0, 0)),
            pl.BlockSpec((1, 3, nhid), lambda i: (0, 0, 0)),
        ],
        out_specs=[
            pl.BlockSpec((tm, nclass), lambda i: (i, 0)),
            pl.BlockSpec((1, 16), lambda i: (0, 0)),
        ],
        out_shape=[
            jax.ShapeDtypeStruct((n, nclass), fdt),
            jax.ShapeDtypeStruct((1, 16), fdt),
        ],
        scratch_shapes=[pltpu.VMEM((3, nclass), fdt)],
        compiler_params=pltpu.CompilerParams(
            dimension_semantics=("arbitrary",)),
    )(adj, v, b2.reshape(1, nclass), xst, hst)

    return (out, mv[0, :12])
